# baseline (speedup 1.0000x reference)
import functools

import jax
import jax.numpy as jnp
from jax import lax
from jax.experimental import pallas as pl
from jax.experimental.pallas import tpu as pltpu
from jax.experimental.pallas import tpu_sc as plsc


def _repack_body(a_ref, b_ref, o_ref):
    o_ref[...] = jnp.concatenate([a_ref[...], b_ref[...]], axis=1)


@functools.lru_cache(maxsize=None)
def _make_repack(N: int, D: int, BR: int):
    H = N // 2
    nb = H // BR
    return pl.pallas_call(
        _repack_body,
        grid=(nb,),
        in_specs=[
            pl.BlockSpec((BR, D), lambda i: (i, 0)),
            pl.BlockSpec((BR, D), lambda i: (i + nb, 0)),
        ],
        out_specs=pl.BlockSpec((BR, 2 * D), lambda i: (i, 0)),
        out_shape=jax.ShapeDtypeStruct((H, 2 * D), jnp.float32),
    )


@functools.lru_cache(maxsize=None)
def _make_gather(B: int, D2: int):
    info = plsc.get_sparse_core_info()
    nc, ns = info.num_cores, info.num_subcores
    nw = nc * ns
    bpw = B // nw
    mesh = plsc.VectorSubcoreMesh(core_axis_name="c", subcore_axis_name="s")

    @functools.partial(
        pl.kernel,
        mesh=mesh,
        out_type=[
            jax.ShapeDtypeStruct((B, D2), jnp.float32),
            jax.ShapeDtypeStruct((B, D2), jnp.float32),
        ],
        scratch_types=[
            pltpu.VMEM((bpw,), jnp.int32),
            pltpu.VMEM((bpw,), jnp.int32),
            pltpu.VMEM((bpw, D2), jnp.float32),
            pltpu.SemaphoreType.DMA,
        ],
    )
    def gather(users_hbm, items_hbm, utab_hbm, itab_hbm, ue_hbm, ie_hbm,
               uidx, iidx, rows, sem):
        wid = lax.axis_index("s") * nc + lax.axis_index("c")
        base = wid * bpw
        pltpu.sync_copy(users_hbm.at[pl.ds(base, bpw)], uidx)
        pltpu.sync_copy(items_hbm.at[pl.ds(base, bpw)], iidx)
        pltpu.async_copy(utab_hbm.at[uidx], rows, sem).wait()
        pltpu.sync_copy(rows, ue_hbm.at[pl.ds(base, bpw)])
        pltpu.async_copy(itab_hbm.at[iidx], rows, sem).wait()
        pltpu.sync_copy(rows, ie_hbm.at[pl.ds(base, bpw)])

    return gather


def _mlp_body(ue_ref, ie_ref, pu_ref, pi_ref, w1u_ref, w1i_ref, b1_ref,
              w2_ref, b2_ref, w3_ref, b3_ref, out_ref):
    D = ue_ref.shape[1] // 2
    pu = pu_ref[...]
    pi = pi_ref[...]
    ue = ue_ref[:, :D] + pu * (ue_ref[:, D:] - ue_ref[:, :D])
    ie = ie_ref[:, :D] + pi * (ie_ref[:, D:] - ie_ref[:, :D])
    h = jnp.dot(ue, w1u_ref[...], preferred_element_type=jnp.float32)
    h = h + jnp.dot(ie, w1i_ref[...], preferred_element_type=jnp.float32)
    h = jnp.maximum(h + b1_ref[...], 0.0)
    h = jnp.dot(h, w2_ref[...], preferred_element_type=jnp.float32)
    h = jnp.maximum(h + b2_ref[...], 0.0)
    o = jnp.dot(h, w3_ref[...], preferred_element_type=jnp.float32)
    out_ref[...] = jax.nn.sigmoid(o + b3_ref[...])


@functools.lru_cache(maxsize=None)
def _make_mlp(B: int, D: int, H1: int, H2: int, BM: int):
    rep = lambda i: (0, 0)
    row = lambda i: (i, 0)
    return pl.pallas_call(
        _mlp_body,
        grid=(B // BM,),
        in_specs=[
            pl.BlockSpec((BM, 2 * D), row),
            pl.BlockSpec((BM, 2 * D), row),
            pl.BlockSpec((BM, 1), row),
            pl.BlockSpec((BM, 1), row),
            pl.BlockSpec((D, H1), rep),
            pl.BlockSpec((D, H1), rep),
            pl.BlockSpec((1, H1), rep),
            pl.BlockSpec((H1, H2), rep),
            pl.BlockSpec((1, H2), rep),
            pl.BlockSpec((H2, 1), rep),
            pl.BlockSpec((1, 1), rep),
        ],
        out_specs=pl.BlockSpec((BM, 1), row),
        out_shape=jax.ShapeDtypeStruct((B, 1), jnp.float32),
    )


def kernel(users, items, user_emb_w, item_emb_w, W1, b1, W2, b2, W3, b3):
    B = users.shape[0]
    N = user_emb_w.shape[0]
    D = user_emb_w.shape[1]
    H1 = W1.shape[0]
    H2 = W2.shape[0]

    users = users.astype(jnp.int32)
    items = items.astype(jnp.int32)

    utabp = _make_repack(N, D, 2000)(user_emb_w, user_emb_w)
    itabp = _make_repack(item_emb_w.shape[0], D, 2000)(item_emb_w, item_emb_w)

    H = N // 2
    Hi = item_emb_w.shape[0] // 2
    ue2, ie2 = _make_gather(B, 2 * D)(users % H, items % Hi, utabp, itabp)

    pu = (users >= H).astype(jnp.float32).reshape(B, 1)
    pi = (items >= Hi).astype(jnp.float32).reshape(B, 1)

    w1u = W1[:, :D].T
    w1i = W1[:, D:].T
    out = _make_mlp(B, D, H1, H2, 2048)(
        ue2, ie2, pu, pi, w1u, w1i, b1.reshape(1, H1), W2.T,
        b2.reshape(1, H2), W3.T, b3.reshape(1, 1))
    return out[:, 0]


# speedup vs baseline: 1.8515x; 1.8515x over previous
import functools

import jax
import jax.numpy as jnp
from jax import lax
from jax.experimental import pallas as pl
from jax.experimental.pallas import tpu as pltpu
from jax.experimental.pallas import tpu_sc as plsc


def _pack_body(a_ref, b_ref, o_ref):
    ta = jnp.transpose(a_ref[...])
    tb = jnp.transpose(b_ref[...])
    o_ref[...] = jnp.concatenate([ta, tb], axis=1)


@functools.lru_cache(maxsize=None)
def _make_pack(N: int, D: int, H: int, BL: int):
    nb = H // BL
    last = (N + BL - 1) // BL - 1
    return pl.pallas_call(
        _pack_body,
        grid=(nb,),
        in_specs=[
            pl.BlockSpec((D, BL), lambda i: (0, i)),
            pl.BlockSpec((D, BL), lambda i: (0, jnp.minimum(i + nb, last))),
        ],
        out_specs=pl.BlockSpec((BL, 2 * D), lambda i: (i, 0)),
        out_shape=jax.ShapeDtypeStruct((H, 2 * D), jnp.float32),
    )


@functools.lru_cache(maxsize=None)
def _make_gather(B: int, D2: int):
    info = plsc.get_sparse_core_info()
    nc, ns = info.num_cores, info.num_subcores
    nw = nc * ns
    bpw = B // nw
    mesh = plsc.VectorSubcoreMesh(core_axis_name="c", subcore_axis_name="s")

    @functools.partial(
        pl.kernel,
        mesh=mesh,
        out_type=[
            jax.ShapeDtypeStruct((B, D2), jnp.float32),
            jax.ShapeDtypeStruct((B, D2), jnp.float32),
        ],
        scratch_types=[
            pltpu.VMEM((bpw,), jnp.int32),
            pltpu.VMEM((bpw,), jnp.int32),
            pltpu.VMEM((bpw, D2), jnp.float32),
            pltpu.SemaphoreType.DMA,
        ],
    )
    def gather(users_hbm, items_hbm, utab_hbm, itab_hbm, ue_hbm, ie_hbm,
               uidx, iidx, rows, sem):
        wid = lax.axis_index("s") * nc + lax.axis_index("c")
        base = wid * bpw
        pltpu.sync_copy(users_hbm.at[pl.ds(base, bpw)], uidx)
        pltpu.sync_copy(items_hbm.at[pl.ds(base, bpw)], iidx)
        pltpu.async_copy(utab_hbm.at[uidx], rows, sem).wait()
        pltpu.sync_copy(rows, ue_hbm.at[pl.ds(base, bpw)])
        pltpu.async_copy(itab_hbm.at[iidx], rows, sem).wait()
        pltpu.sync_copy(rows, ie_hbm.at[pl.ds(base, bpw)])

    return gather


def _mlp_body(ue_ref, ie_ref, pu_ref, pi_ref, w1u_ref, w1i_ref, b1_ref,
              w2_ref, b2_ref, w3_ref, b3_ref, out_ref):
    D = ue_ref.shape[1] // 2
    pu = pu_ref[...]
    pi = pi_ref[...]
    ue = ue_ref[:, :D] + pu * (ue_ref[:, D:] - ue_ref[:, :D])
    ie = ie_ref[:, :D] + pi * (ie_ref[:, D:] - ie_ref[:, :D])
    h = jnp.dot(ue, w1u_ref[...], preferred_element_type=jnp.float32)
    h = h + jnp.dot(ie, w1i_ref[...], preferred_element_type=jnp.float32)
    h = jnp.maximum(h + b1_ref[...], 0.0)
    h = jnp.dot(h, w2_ref[...], preferred_element_type=jnp.float32)
    h = jnp.maximum(h + b2_ref[...], 0.0)
    o = jnp.dot(h, w3_ref[...], preferred_element_type=jnp.float32)
    out_ref[...] = jax.nn.sigmoid(o + b3_ref[...])


@functools.lru_cache(maxsize=None)
def _make_mlp(B: int, D: int, H1: int, H2: int, BM: int):
    rep = lambda i: (0, 0)
    row = lambda i: (i, 0)
    return pl.pallas_call(
        _mlp_body,
        grid=(B // BM,),
        in_specs=[
            pl.BlockSpec((BM, 2 * D), row),
            pl.BlockSpec((BM, 2 * D), row),
            pl.BlockSpec((BM, 1), row),
            pl.BlockSpec((BM, 1), row),
            pl.BlockSpec((D, H1), rep),
            pl.BlockSpec((D, H1), rep),
            pl.BlockSpec((1, H1), rep),
            pl.BlockSpec((H1, H2), rep),
            pl.BlockSpec((1, H2), rep),
            pl.BlockSpec((H2, 1), rep),
            pl.BlockSpec((1, 1), rep),
        ],
        out_specs=pl.BlockSpec((BM, 1), row),
        out_shape=jax.ShapeDtypeStruct((B, 1), jnp.float32),
    )


def kernel(users, items, user_emb_w, item_emb_w, W1, b1, W2, b2, W3, b3):
    B = users.shape[0]
    N = user_emb_w.shape[0]
    D = user_emb_w.shape[1]
    H1 = W1.shape[0]
    H2 = W2.shape[0]

    users = users.astype(jnp.int32)
    items = items.astype(jnp.int32)

    utabT = user_emb_w.T
    itabT = item_emb_w.T
    H = 1 << (N - 1).bit_length() >> 1
    Hi = 1 << (item_emb_w.shape[0] - 1).bit_length() >> 1
    utabp = _make_pack(N, D, H, 2048)(utabT, utabT)
    itabp = _make_pack(item_emb_w.shape[0], D, Hi, 2048)(itabT, itabT)

    ue2, ie2 = _make_gather(B, 2 * D)(users % H, items % Hi, utabp, itabp)

    pu = (users >= H).astype(jnp.float32).reshape(B, 1)
    pi = (items >= Hi).astype(jnp.float32).reshape(B, 1)

    w1u = W1[:, :D].T
    w1i = W1[:, D:].T
    out = _make_mlp(B, D, H1, H2, 2048)(
        ue2, ie2, pu, pi, w1u, w1i, b1.reshape(1, H1), W2.T,
        b2.reshape(1, H2), W3.T, b3.reshape(1, 1))
    return out[:, 0]


# speedup vs baseline: 2.2878x; 1.2356x over previous
import functools

import jax
import jax.numpy as jnp
from jax import lax
from jax.experimental import pallas as pl
from jax.experimental.pallas import tpu as pltpu
from jax.experimental.pallas import tpu_sc as plsc


def _pack_body(a_ref, b_ref, o_ref):
    ta = jnp.transpose(a_ref[...])
    tb = jnp.transpose(b_ref[...])
    o_ref[...] = jnp.concatenate([ta, tb], axis=1)


@functools.lru_cache(maxsize=None)
def _make_pack(N: int, D: int, H: int, BL: int):
    nb = H // BL
    last = (N + BL - 1) // BL - 1
    return pl.pallas_call(
        _pack_body,
        grid=(nb,),
        in_specs=[
            pl.BlockSpec((D, BL), lambda i: (0, i)),
            pl.BlockSpec((D, BL), lambda i: (0, jnp.minimum(i + nb, last))),
        ],
        out_specs=pl.BlockSpec((BL, 2 * D), lambda i: (i, 0)),
        out_shape=jax.ShapeDtypeStruct((H, 2 * D), jnp.float32),
    )


@functools.lru_cache(maxsize=None)
def _make_gather(B: int, D2: int):
    info = plsc.get_sparse_core_info()
    nc, ns = info.num_cores, info.num_subcores
    nw = nc * ns
    bpw = B // nw
    mesh = plsc.VectorSubcoreMesh(core_axis_name="c", subcore_axis_name="s")

    @functools.partial(
        pl.kernel,
        mesh=mesh,
        out_type=[
            jax.ShapeDtypeStruct((B, D2), jnp.float32),
            jax.ShapeDtypeStruct((B, D2), jnp.float32),
        ],
        scratch_types=[
            pltpu.VMEM((bpw,), jnp.int32),
            pltpu.VMEM((bpw,), jnp.int32),
            pltpu.VMEM((bpw, D2), jnp.float32),
            pltpu.SemaphoreType.DMA,
        ],
    )
    def gather(users_hbm, items_hbm, utab_hbm, itab_hbm, ue_hbm, ie_hbm,
               uidx, iidx, rows, sem):
        wid = lax.axis_index("s") * nc + lax.axis_index("c")
        base = wid * bpw
        pltpu.sync_copy(users_hbm.at[pl.ds(base, bpw)], uidx)
        pltpu.sync_copy(items_hbm.at[pl.ds(base, bpw)], iidx)
        pltpu.async_copy(utab_hbm.at[uidx], rows, sem).wait()
        pltpu.sync_copy(rows, ue_hbm.at[pl.ds(base, bpw)])
        pltpu.async_copy(itab_hbm.at[iidx], rows, sem).wait()
        pltpu.sync_copy(rows, ie_hbm.at[pl.ds(base, bpw)])

    return gather


def _mlp_body(ue_ref, ie_ref, pu_ref, pi_ref, w1u_ref, w1i_ref, b1_ref,
              w2_ref, b2_ref, w3_ref, b3_ref, out_ref):
    D = ue_ref.shape[1] // 2
    pu = pu_ref[...]
    pi = pi_ref[...]
    ue = ue_ref[:, :D] + pu * (ue_ref[:, D:] - ue_ref[:, :D])
    ie = ie_ref[:, :D] + pi * (ie_ref[:, D:] - ie_ref[:, :D])
    h = jnp.dot(ue, w1u_ref[...], preferred_element_type=jnp.float32)
    h = h + jnp.dot(ie, w1i_ref[...], preferred_element_type=jnp.float32)
    h = jnp.maximum(h + b1_ref[...], 0.0)
    h = jnp.dot(h, w2_ref[...], preferred_element_type=jnp.float32)
    h = jnp.maximum(h + b2_ref[...], 0.0)
    o = jnp.dot(h, w3_ref[...], preferred_element_type=jnp.float32)
    out_ref[...] = jax.nn.sigmoid(o + b3_ref[...])


@functools.lru_cache(maxsize=None)
def _make_mlp(B: int, D: int, H1: int, H2: int, BM: int):
    rep = lambda i: (0, 0)
    row = lambda i: (i, 0)
    return pl.pallas_call(
        _mlp_body,
        grid=(B // BM,),
        in_specs=[
            pl.BlockSpec((BM, 2 * D), row),
            pl.BlockSpec((BM, 2 * D), row),
            pl.BlockSpec((BM, 1), row),
            pl.BlockSpec((BM, 1), row),
            pl.BlockSpec((D, H1), rep),
            pl.BlockSpec((D, H1), rep),
            pl.BlockSpec((1, H1), rep),
            pl.BlockSpec((H1, H2), rep),
            pl.BlockSpec((1, H2), rep),
            pl.BlockSpec((H2, 1), rep),
            pl.BlockSpec((1, 1), rep),
        ],
        out_specs=pl.BlockSpec((BM, 1), row),
        out_shape=jax.ShapeDtypeStruct((B, 1), jnp.float32),
    )


def kernel(users, items, user_emb_w, item_emb_w, W1, b1, W2, b2, W3, b3):
    B = users.shape[0]
    N = user_emb_w.shape[0]
    D = user_emb_w.shape[1]
    H1 = W1.shape[0]
    H2 = W2.shape[0]

    users = users.astype(jnp.int32)
    items = items.astype(jnp.int32)

    utabT = user_emb_w.T
    itabT = item_emb_w.T
    H = 1 << (N - 1).bit_length() >> 1
    Hi = 1 << (item_emb_w.shape[0] - 1).bit_length() >> 1
    utabp = _make_pack(N, D, H, 4096)(utabT, utabT)
    itabp = _make_pack(item_emb_w.shape[0], D, Hi, 4096)(itabT, itabT)

    ue2, ie2 = _make_gather(B, 2 * D)(users % H, items % Hi, utabp, itabp)

    pu = (users >= H).astype(jnp.float32).reshape(B, 1)
    pi = (items >= Hi).astype(jnp.float32).reshape(B, 1)

    w1u = W1[:, :D].T
    w1i = W1[:, D:].T
    out = _make_mlp(B, D, H1, H2, 2048)(
        ue2, ie2, pu, pi, w1u, w1i, b1.reshape(1, H1), W2.T,
        b2.reshape(1, H2), W3.T, b3.reshape(1, 1))
    return out[:, 0]


# speedup vs baseline: 2.5762x; 1.1261x over previous
import functools

import jax
import jax.numpy as jnp
from jax import lax
from jax.experimental import pallas as pl
from jax.experimental.pallas import tpu as pltpu
from jax.experimental.pallas import tpu_sc as plsc


def _pack_body(a_ref, b_ref, o_ref):
    ta = jnp.transpose(a_ref[...])
    tb = jnp.transpose(b_ref[...])
    o_ref[...] = jnp.concatenate([ta, tb], axis=1)


@functools.lru_cache(maxsize=None)
def _make_pack(N: int, D: int, H: int, BL: int):
    nb = H // BL
    last = (N + BL - 1) // BL - 1
    return pl.pallas_call(
        _pack_body,
        grid=(nb,),
        in_specs=[
            pl.BlockSpec((D, BL), lambda i: (0, i)),
            pl.BlockSpec((D, BL), lambda i: (0, jnp.minimum(i + nb, last))),
        ],
        out_specs=pl.BlockSpec((BL, 2 * D), lambda i: (i, 0)),
        out_shape=jax.ShapeDtypeStruct((H, 2 * D), jnp.float32),
    )


@functools.lru_cache(maxsize=None)
def _make_gather(B: int, D2: int):
    info = plsc.get_sparse_core_info()
    nc, ns = info.num_cores, info.num_subcores
    nw = nc * ns
    bpw = B // nw
    mesh = plsc.VectorSubcoreMesh(core_axis_name="c", subcore_axis_name="s")

    @functools.partial(
        pl.kernel,
        mesh=mesh,
        out_type=[
            jax.ShapeDtypeStruct((B, D2), jnp.float32),
            jax.ShapeDtypeStruct((B, D2), jnp.float32),
        ],
        scratch_types=[
            pltpu.VMEM((bpw,), jnp.int32),
            pltpu.VMEM((bpw,), jnp.int32),
            pltpu.VMEM((bpw, D2), jnp.float32),
            pltpu.SemaphoreType.DMA,
        ],
    )
    def gather(users_hbm, items_hbm, utab_hbm, itab_hbm, ue_hbm, ie_hbm,
               uidx, iidx, rows, sem):
        wid = lax.axis_index("s") * nc + lax.axis_index("c")
        base = wid * bpw
        pltpu.sync_copy(users_hbm.at[pl.ds(base, bpw)], uidx)
        pltpu.sync_copy(items_hbm.at[pl.ds(base, bpw)], iidx)
        pltpu.async_copy(utab_hbm.at[uidx], rows, sem).wait()
        pltpu.sync_copy(rows, ue_hbm.at[pl.ds(base, bpw)])
        pltpu.async_copy(itab_hbm.at[iidx], rows, sem).wait()
        pltpu.sync_copy(rows, ie_hbm.at[pl.ds(base, bpw)])

    return gather


def _mlp_body(ue_ref, ie_ref, pu_ref, pi_ref, w1u_ref, w1i_ref, b1_ref,
              w2_ref, b2_ref, w3_ref, b3_ref, out_ref):
    D = ue_ref.shape[1] // 2
    pu = pu_ref[...]
    pi = pi_ref[...]
    ue = ue_ref[:, :D] + pu * (ue_ref[:, D:] - ue_ref[:, :D])
    ie = ie_ref[:, :D] + pi * (ie_ref[:, D:] - ie_ref[:, :D])
    h = jnp.dot(ue, w1u_ref[...], preferred_element_type=jnp.float32)
    h = h + jnp.dot(ie, w1i_ref[...], preferred_element_type=jnp.float32)
    h = jnp.maximum(h + b1_ref[...], 0.0)
    h = jnp.dot(h, w2_ref[...], preferred_element_type=jnp.float32)
    h = jnp.maximum(h + b2_ref[...], 0.0)
    o = jnp.dot(h, w3_ref[...], preferred_element_type=jnp.float32)
    out_ref[...] = jax.nn.sigmoid(o + b3_ref[...])


@functools.lru_cache(maxsize=None)
def _make_mlp(B: int, D: int, H1: int, H2: int, BM: int):
    rep = lambda i: (0, 0)
    row = lambda i: (i, 0)
    return pl.pallas_call(
        _mlp_body,
        grid=(B // BM,),
        in_specs=[
            pl.BlockSpec((BM, 2 * D), row),
            pl.BlockSpec((BM, 2 * D), row),
            pl.BlockSpec((BM, 1), row),
            pl.BlockSpec((BM, 1), row),
            pl.BlockSpec((D, H1), rep),
            pl.BlockSpec((D, H1), rep),
            pl.BlockSpec((1, H1), rep),
            pl.BlockSpec((H1, H2), rep),
            pl.BlockSpec((1, H2), rep),
            pl.BlockSpec((H2, 1), rep),
            pl.BlockSpec((1, 1), rep),
        ],
        out_specs=pl.BlockSpec((BM, 1), row),
        out_shape=jax.ShapeDtypeStruct((B, 1), jnp.float32),
    )


def kernel(users, items, user_emb_w, item_emb_w, W1, b1, W2, b2, W3, b3):
    B = users.shape[0]
    N = user_emb_w.shape[0]
    D = user_emb_w.shape[1]
    H1 = W1.shape[0]
    H2 = W2.shape[0]

    users = users.astype(jnp.int32)
    items = items.astype(jnp.int32)

    utabT = user_emb_w.T
    itabT = item_emb_w.T
    H = 1 << (N - 1).bit_length() >> 1
    Hi = 1 << (item_emb_w.shape[0] - 1).bit_length() >> 1
    utabp = _make_pack(N, D, H, 8192)(utabT, utabT)
    itabp = _make_pack(item_emb_w.shape[0], D, Hi, 8192)(itabT, itabT)

    ue2, ie2 = _make_gather(B, 2 * D)(users % H, items % Hi, utabp, itabp)

    pu = (users >= H).astype(jnp.float32).reshape(B, 1)
    pi = (items >= Hi).astype(jnp.float32).reshape(B, 1)

    w1u = W1[:, :D].T
    w1i = W1[:, D:].T
    out = _make_mlp(B, D, H1, H2, 2048)(
        ue2, ie2, pu, pi, w1u, w1i, b1.reshape(1, H1), W2.T,
        b2.reshape(1, H2), W3.T, b3.reshape(1, 1))
    return out[:, 0]


# speedup vs baseline: 2.7320x; 1.0605x over previous
import functools

import jax
import jax.numpy as jnp
from jax import lax
from jax.experimental import pallas as pl
from jax.experimental.pallas import tpu as pltpu
from jax.experimental.pallas import tpu_sc as plsc


def _pack_body(a_ref, b_ref, o_ref):
    ta = jnp.transpose(a_ref[...])
    tb = jnp.transpose(b_ref[...])
    o_ref[...] = jnp.concatenate([ta, tb], axis=1)


@functools.lru_cache(maxsize=None)
def _make_pack(N: int, D: int, H: int, BL: int):
    nb = H // BL
    last = (N + BL - 1) // BL - 1
    return pl.pallas_call(
        _pack_body,
        grid=(nb,),
        in_specs=[
            pl.BlockSpec((D, BL), lambda i: (0, i)),
            pl.BlockSpec((D, BL), lambda i: (0, jnp.minimum(i + nb, last))),
        ],
        out_specs=pl.BlockSpec((BL, 2 * D), lambda i: (i, 0)),
        out_shape=jax.ShapeDtypeStruct((H, 2 * D), jnp.float32),
    )


@functools.lru_cache(maxsize=None)
def _make_gather(B: int, D2: int):
    info = plsc.get_sparse_core_info()
    nc, ns = info.num_cores, info.num_subcores
    nw = nc * ns
    bpw = B // nw
    mesh = plsc.VectorSubcoreMesh(core_axis_name="c", subcore_axis_name="s")

    @functools.partial(
        pl.kernel,
        mesh=mesh,
        out_type=[
            jax.ShapeDtypeStruct((B, D2), jnp.float32),
            jax.ShapeDtypeStruct((B, D2), jnp.float32),
        ],
        scratch_types=[
            pltpu.VMEM((bpw,), jnp.int32),
            pltpu.VMEM((bpw,), jnp.int32),
            pltpu.VMEM((bpw, D2), jnp.float32),
            pltpu.SemaphoreType.DMA,
        ],
    )
    def gather(users_hbm, items_hbm, utab_hbm, itab_hbm, ue_hbm, ie_hbm,
               uidx, iidx, rows, sem):
        wid = lax.axis_index("s") * nc + lax.axis_index("c")
        base = wid * bpw
        pltpu.sync_copy(users_hbm.at[pl.ds(base, bpw)], uidx)
        pltpu.sync_copy(items_hbm.at[pl.ds(base, bpw)], iidx)
        pltpu.async_copy(utab_hbm.at[uidx], rows, sem).wait()
        pltpu.sync_copy(rows, ue_hbm.at[pl.ds(base, bpw)])
        pltpu.async_copy(itab_hbm.at[iidx], rows, sem).wait()
        pltpu.sync_copy(rows, ie_hbm.at[pl.ds(base, bpw)])

    return gather


def _mlp_body(ue_ref, ie_ref, pu_ref, pi_ref, w1u_ref, w1i_ref, b1_ref,
              w2_ref, b2_ref, w3_ref, b3_ref, out_ref):
    D = ue_ref.shape[1] // 2
    pu = pu_ref[...]
    pi = pi_ref[...]
    ue = ue_ref[:, :D] + pu * (ue_ref[:, D:] - ue_ref[:, :D])
    ie = ie_ref[:, :D] + pi * (ie_ref[:, D:] - ie_ref[:, :D])
    h = jnp.dot(ue, w1u_ref[...], preferred_element_type=jnp.float32)
    h = h + jnp.dot(ie, w1i_ref[...], preferred_element_type=jnp.float32)
    h = jnp.maximum(h + b1_ref[...], 0.0)
    h = jnp.dot(h, w2_ref[...], preferred_element_type=jnp.float32)
    h = jnp.maximum(h + b2_ref[...], 0.0)
    o = jnp.dot(h, w3_ref[...], preferred_element_type=jnp.float32)
    out_ref[...] = jax.nn.sigmoid(o + b3_ref[...])


@functools.lru_cache(maxsize=None)
def _make_mlp(B: int, D: int, H1: int, H2: int, BM: int):
    rep = lambda i: (0, 0)
    row = lambda i: (i, 0)
    return pl.pallas_call(
        _mlp_body,
        grid=(B // BM,),
        in_specs=[
            pl.BlockSpec((BM, 2 * D), row),
            pl.BlockSpec((BM, 2 * D), row),
            pl.BlockSpec((BM, 1), row),
            pl.BlockSpec((BM, 1), row),
            pl.BlockSpec((D, H1), rep),
            pl.BlockSpec((D, H1), rep),
            pl.BlockSpec((1, H1), rep),
            pl.BlockSpec((H1, H2), rep),
            pl.BlockSpec((1, H2), rep),
            pl.BlockSpec((H2, 1), rep),
            pl.BlockSpec((1, 1), rep),
        ],
        out_specs=pl.BlockSpec((BM, 1), row),
        out_shape=jax.ShapeDtypeStruct((B, 1), jnp.float32),
    )


def kernel(users, items, user_emb_w, item_emb_w, W1, b1, W2, b2, W3, b3):
    B = users.shape[0]
    N = user_emb_w.shape[0]
    D = user_emb_w.shape[1]
    H1 = W1.shape[0]
    H2 = W2.shape[0]

    users = users.astype(jnp.int32)
    items = items.astype(jnp.int32)

    utabT = user_emb_w.T
    itabT = item_emb_w.T
    H = 1 << (N - 1).bit_length() >> 1
    Hi = 1 << (item_emb_w.shape[0] - 1).bit_length() >> 1
    utabp = _make_pack(N, D, H, 16384)(utabT, utabT)
    itabp = _make_pack(item_emb_w.shape[0], D, Hi, 16384)(itabT, itabT)

    ue2, ie2 = _make_gather(B, 2 * D)(users % H, items % Hi, utabp, itabp)

    pu = (users >= H).astype(jnp.float32).reshape(B, 1)
    pi = (items >= Hi).astype(jnp.float32).reshape(B, 1)

    w1u = W1[:, :D].T
    w1i = W1[:, D:].T
    out = _make_mlp(B, D, H1, H2, 2048)(
        ue2, ie2, pu, pi, w1u, w1i, b1.reshape(1, H1), W2.T,
        b2.reshape(1, H2), W3.T, b3.reshape(1, 1))
    return out[:, 0]


# speedup vs baseline: 2.7351x; 1.0011x over previous
import functools

import jax
import jax.numpy as jnp
from jax import lax
from jax.experimental import pallas as pl
from jax.experimental.pallas import tpu as pltpu
from jax.experimental.pallas import tpu_sc as plsc


def _pack_body(a_ref, b_ref, o_ref):
    eye = jnp.eye(a_ref.shape[0], dtype=jnp.float32)
    dn = (((0,), (0,)), ((), ()))
    ta = lax.dot_general(a_ref[...], eye, dn,
                         preferred_element_type=jnp.float32)
    tb = lax.dot_general(b_ref[...], eye, dn,
                         preferred_element_type=jnp.float32)
    o_ref[...] = jnp.concatenate([ta, tb], axis=1)


@functools.lru_cache(maxsize=None)
def _make_pack(N: int, D: int, H: int, BL: int):
    nb = H // BL
    last = (N + BL - 1) // BL - 1
    return pl.pallas_call(
        _pack_body,
        grid=(nb,),
        in_specs=[
            pl.BlockSpec((D, BL), lambda i: (0, i)),
            pl.BlockSpec((D, BL), lambda i: (0, jnp.minimum(i + nb, last))),
        ],
        out_specs=pl.BlockSpec((BL, 2 * D), lambda i: (i, 0)),
        out_shape=jax.ShapeDtypeStruct((H, 2 * D), jnp.float32),
    )


@functools.lru_cache(maxsize=None)
def _make_gather(B: int, D2: int):
    info = plsc.get_sparse_core_info()
    nc, ns = info.num_cores, info.num_subcores
    nw = nc * ns
    bpw = B // nw
    mesh = plsc.VectorSubcoreMesh(core_axis_name="c", subcore_axis_name="s")

    @functools.partial(
        pl.kernel,
        mesh=mesh,
        out_type=[
            jax.ShapeDtypeStruct((B, D2), jnp.float32),
            jax.ShapeDtypeStruct((B, D2), jnp.float32),
        ],
        scratch_types=[
            pltpu.VMEM((bpw,), jnp.int32),
            pltpu.VMEM((bpw,), jnp.int32),
            pltpu.VMEM((bpw, D2), jnp.float32),
            pltpu.SemaphoreType.DMA,
        ],
    )
    def gather(users_hbm, items_hbm, utab_hbm, itab_hbm, ue_hbm, ie_hbm,
               uidx, iidx, rows, sem):
        wid = lax.axis_index("s") * nc + lax.axis_index("c")
        base = wid * bpw
        pltpu.sync_copy(users_hbm.at[pl.ds(base, bpw)], uidx)
        pltpu.sync_copy(items_hbm.at[pl.ds(base, bpw)], iidx)
        pltpu.async_copy(utab_hbm.at[uidx], rows, sem).wait()
        pltpu.sync_copy(rows, ue_hbm.at[pl.ds(base, bpw)])
        pltpu.async_copy(itab_hbm.at[iidx], rows, sem).wait()
        pltpu.sync_copy(rows, ie_hbm.at[pl.ds(base, bpw)])

    return gather


def _mlp_body(ue_ref, ie_ref, pu_ref, pi_ref, w1u_ref, w1i_ref, b1_ref,
              w2_ref, b2_ref, w3_ref, b3_ref, out_ref):
    D = ue_ref.shape[1] // 2
    pu = pu_ref[...]
    pi = pi_ref[...]
    ue = ue_ref[:, :D] + pu * (ue_ref[:, D:] - ue_ref[:, :D])
    ie = ie_ref[:, :D] + pi * (ie_ref[:, D:] - ie_ref[:, :D])
    h = jnp.dot(ue, w1u_ref[...], preferred_element_type=jnp.float32)
    h = h + jnp.dot(ie, w1i_ref[...], preferred_element_type=jnp.float32)
    h = jnp.maximum(h + b1_ref[...], 0.0)
    h = jnp.dot(h, w2_ref[...], preferred_element_type=jnp.float32)
    h = jnp.maximum(h + b2_ref[...], 0.0)
    o = jnp.dot(h, w3_ref[...], preferred_element_type=jnp.float32)
    out_ref[...] = jax.nn.sigmoid(o + b3_ref[...])


@functools.lru_cache(maxsize=None)
def _make_mlp(B: int, D: int, H1: int, H2: int, BM: int):
    rep = lambda i: (0, 0)
    row = lambda i: (i, 0)
    return pl.pallas_call(
        _mlp_body,
        grid=(B // BM,),
        in_specs=[
            pl.BlockSpec((BM, 2 * D), row),
            pl.BlockSpec((BM, 2 * D), row),
            pl.BlockSpec((BM, 1), row),
            pl.BlockSpec((BM, 1), row),
            pl.BlockSpec((D, H1), rep),
            pl.BlockSpec((D, H1), rep),
            pl.BlockSpec((1, H1), rep),
            pl.BlockSpec((H1, H2), rep),
            pl.BlockSpec((1, H2), rep),
            pl.BlockSpec((H2, 1), rep),
            pl.BlockSpec((1, 1), rep),
        ],
        out_specs=pl.BlockSpec((BM, 1), row),
        out_shape=jax.ShapeDtypeStruct((B, 1), jnp.float32),
    )


def kernel(users, items, user_emb_w, item_emb_w, W1, b1, W2, b2, W3, b3):
    B = users.shape[0]
    N = user_emb_w.shape[0]
    D = user_emb_w.shape[1]
    H1 = W1.shape[0]
    H2 = W2.shape[0]

    users = users.astype(jnp.int32)
    items = items.astype(jnp.int32)

    utabT = user_emb_w.T
    itabT = item_emb_w.T
    H = 1 << (N - 1).bit_length() >> 1
    Hi = 1 << (item_emb_w.shape[0] - 1).bit_length() >> 1
    utabp = _make_pack(N, D, H, 16384)(utabT, utabT)
    itabp = _make_pack(item_emb_w.shape[0], D, Hi, 16384)(itabT, itabT)

    ue2, ie2 = _make_gather(B, 2 * D)(users % H, items % Hi, utabp, itabp)

    pu = (users >= H).astype(jnp.float32).reshape(B, 1)
    pi = (items >= Hi).astype(jnp.float32).reshape(B, 1)

    w1u = W1[:, :D].T
    w1i = W1[:, D:].T
    out = _make_mlp(B, D, H1, H2, 2048)(
        ue2, ie2, pu, pi, w1u, w1i, b1.reshape(1, H1), W2.T,
        b2.reshape(1, H2), W3.T, b3.reshape(1, 1))
    return out[:, 0]
